# baseline (speedup 1.0000x reference)
import jax
import jax.numpy as jnp
from jax.experimental import pallas as pl
from jax.experimental.pallas import tpu as pltpu


def _add_kernel(x_hbm, p_hbm, o_hbm, xbuf, obuf, pbuf, xsem, psem, osem):
    nb = x_hbm.shape[0] // 2

    pltpu.make_async_copy(p_hbm, pbuf, psem).start()
    for i in range(nb):
        pltpu.make_async_copy(
            x_hbm.at[pl.ds(2 * i, 2)], xbuf.at[i], xsem.at[i]
        ).start()
    pltpu.make_async_copy(p_hbm, pbuf, psem).wait()

    for i in range(nb):
        pltpu.make_async_copy(
            x_hbm.at[pl.ds(2 * i, 2)], xbuf.at[i], xsem.at[i]
        ).wait()
        for j in range(2):
            obuf[i, j] = xbuf[i, j] + pbuf[...]
            pltpu.make_async_copy(
                obuf.at[i, j], o_hbm.at[2 * i + j], osem.at[i, j]
            ).start()

    for i in range(nb):
        for j in range(2):
            pltpu.make_async_copy(
                obuf.at[i, j], o_hbm.at[2 * i + j], osem.at[i, j]
            ).wait()


def kernel(x, pos_table):
    B, T, D = x.shape
    return pl.pallas_call(
        _add_kernel,
        in_specs=[
            pl.BlockSpec(memory_space=pl.ANY),
            pl.BlockSpec(memory_space=pl.ANY),
        ],
        out_specs=pl.BlockSpec(memory_space=pl.ANY),
        out_shape=jax.ShapeDtypeStruct((B, T, D), x.dtype),
        scratch_shapes=[
            pltpu.VMEM((B // 2, 2, T, D), x.dtype),
            pltpu.VMEM((B // 2, 2, T, D), x.dtype),
            pltpu.VMEM((T, D), x.dtype),
            pltpu.SemaphoreType.DMA((B // 2,)),
            pltpu.SemaphoreType.DMA,
            pltpu.SemaphoreType.DMA((B // 2, 2)),
        ],
    )(x, pos_table)
